# baseline (speedup 1.0000x reference)
import functools

import jax
import jax.numpy as jnp
from jax import lax
from jax.experimental import pallas as pl
from jax.experimental.pallas import tpu as pltpu
from jax.experimental.pallas import tpu_sc as plsc

TOP_K = 8


def _routing_body(cls_ref, key_ref, ids_ref, rs_ref):
    eps = 1e-12
    k = key_ref[...]
    kn = jnp.sqrt(jnp.sum(k * k, axis=1, keepdims=True))
    k_n = k / jnp.maximum(kn, eps)
    x = cls_ref[...]
    xn = jnp.sqrt(jnp.sum(x * x, axis=1, keepdims=True))
    x_n = x / jnp.maximum(xn, eps)
    sim0 = jax.lax.dot_general(
        x_n, k_n, (((1,), (1,)), ((), ())),
        preferred_element_type=jnp.float32)
    B, P = sim0.shape

    col = jax.lax.broadcasted_iota(jnp.int32, (B, P), 1)
    sim = sim0
    counts2d = jnp.zeros((B, P), jnp.int32)
    for _ in range(TOP_K):
        m = jnp.max(sim, axis=1, keepdims=True)
        cand = jnp.where(sim == m, col, P)
        j = jnp.min(cand, axis=1, keepdims=True)
        oh = col == j
        counts2d = counts2d + oh.astype(jnp.int32)
        sim = jnp.where(oh, -jnp.inf, sim)

    cnt = jnp.sum(counts2d, axis=0, keepdims=True)
    p_row = jax.lax.broadcasted_iota(jnp.int32, (1, P), 1)
    key2 = (cnt * (2 * P) + (P - 1 - p_row)).astype(jnp.float32)
    ri = jax.lax.broadcasted_iota(jnp.int32, (P, P), 0)
    ci = jax.lax.broadcasted_iota(jnp.int32, (P, P), 1)
    ident = (ri == ci).astype(jnp.float32)
    key2_col = jax.lax.dot_general(
        ident, key2, (((1,), (1,)), ((), ())),
        preferred_element_type=jnp.float32)
    gt = (key2_col > key2).astype(jnp.int32)
    rank = jnp.sum(gt, axis=0, keepdims=True)
    for j in range(TOP_K):
        ids_ref[0, j] = jnp.sum(jnp.where(rank == j, p_row, 0))
    colsum = jnp.sum(sim0, axis=0, keepdims=True)
    sel = (rank < TOP_K).astype(jnp.float32)
    rs_ref[0, 0] = jnp.sum(colsum * sel) / B


def _routing(cls_features, prompt_key):
    return pl.pallas_call(
        _routing_body,
        out_shape=(
            jax.ShapeDtypeStruct((1, TOP_K), jnp.int32),
            jax.ShapeDtypeStruct((1, 1), jnp.float32),
        ),
        in_specs=[
            pl.BlockSpec(memory_space=pltpu.VMEM),
            pl.BlockSpec(memory_space=pltpu.VMEM),
        ],
        out_specs=(
            pl.BlockSpec(memory_space=pltpu.SMEM),
            pl.BlockSpec(memory_space=pltpu.SMEM),
        ),
    )(cls_features, prompt_key)


def _make_sc_gather(L, B, LEN, C, n_workers, cells_per_w):
    D = LEN * C
    info = plsc.get_sparse_core_info()
    nc = info.num_cores
    mesh = plsc.VectorSubcoreMesh(core_axis_name="c", subcore_axis_name="s")

    @functools.partial(
        pl.kernel,
        mesh=mesh,
        out_type=jax.ShapeDtypeStruct((L * B, TOP_K, D), jnp.float32),
        scratch_types=[
            pltpu.VMEM((L * TOP_K,), jnp.int32),
            pltpu.VMEM((TOP_K, D), jnp.float32),
            pltpu.SemaphoreType.DMA,
            pltpu.SemaphoreType.DMA,
        ],
    )
    def sc_gather(table_hbm, idx_hbm, out_hbm, idx_v, slab, rsem, wsem):
        wid = lax.axis_index("s") * nc + lax.axis_index("c")
        pltpu.sync_copy(idx_hbm, idx_v)

        def gather_slab(layer):
            off = pl.multiple_of(layer * TOP_K, 8)
            pltpu.async_copy(
                table_hbm.at[idx_v.at[pl.ds(off, TOP_K)]],
                slab, rsem).wait()

        def fire(m, _):
            pltpu.async_copy(slab, out_hbm.at[m], wsem)
            return _

        def drain(m, _):
            pltpu.make_async_copy(slab, out_hbm.at[m], wsem).wait()
            return _

        m0 = wid * cells_per_w
        lA = m0 // B
        nA = jnp.minimum(B - (m0 - lA * B), cells_per_w)
        gather_slab(lA)
        lax.fori_loop(m0, m0 + nA, fire, 0)
        lax.fori_loop(m0, m0 + nA, drain, 0)

        @pl.when(nA < cells_per_w)
        def _():
            gather_slab(lA + 1)
            lax.fori_loop(m0 + nA, m0 + cells_per_w, fire, 0)
            lax.fori_loop(m0 + nA, m0 + cells_per_w, drain, 0)

    return sc_gather


def kernel(x_embed, cls_features, prompt, prompt_key):
    B = x_embed.shape[0]
    L, P, LEN, C = prompt.shape
    n_workers = 32
    cells_per_w = (L * B) // n_workers

    ids, rs = _routing(cls_features, prompt_key)

    larr = jnp.arange(L * TOP_K, dtype=jnp.int32)
    idx_flat = (larr // TOP_K) * P + ids[0, larr % TOP_K]
    idx_flat = idx_flat.astype(jnp.int32)

    table = prompt.reshape(L * P, LEN * C)
    sc_gather = _make_sc_gather(L, B, LEN, C, n_workers, cells_per_w)
    out_flat = sc_gather(table, idx_flat)
    batched_prompt = out_flat.reshape(L, B, TOP_K * LEN, C)

    return batched_prompt, rs.reshape(())


# speedup vs baseline: 3.5158x; 3.5158x over previous
import functools

import jax
import jax.numpy as jnp
from jax import lax
from jax.experimental import pallas as pl
from jax.experimental.pallas import tpu as pltpu
from jax.experimental.pallas import tpu_sc as plsc

TOP_K = 8


def _sim_body(cls_ref, key_ref, sim_ref):
    eps = 1e-12
    k = key_ref[...]
    kn = jnp.sqrt(jnp.sum(k * k, axis=1, keepdims=True))
    k_n = k / jnp.maximum(kn, eps)
    x = cls_ref[...]
    xn = jnp.sqrt(jnp.sum(x * x, axis=1, keepdims=True))
    x_n = x / jnp.maximum(xn, eps)
    sim_ref[...] = jax.lax.dot_general(
        x_n, k_n, (((1,), (1,)), ((), ())),
        preferred_element_type=jnp.float32)


def _rank64(vregs, strict_only):
    ii = lax.broadcasted_iota(jnp.int32, (16,), 0)
    cnts = [jnp.zeros((16,), jnp.int32) for _ in range(4)]
    for pp in range(64):
        sp = jnp.zeros((16,), vregs[0].dtype) + vregs[pp // 16][pp % 16]
        for g in range(4):
            p_vec = ii + 16 * g
            gtm = (sp > vregs[g]).astype(jnp.int32)
            if strict_only:
                cnts[g] = cnts[g] + gtm
            else:
                eqm = ((sp == vregs[g]) & (pp < p_vec)).astype(jnp.int32)
                cnts[g] = cnts[g] + gtm + eqm
    return cnts


def _sc_route_body(sim_hbm, ids_out, rs_out,
                   myrows, myflat, mymask, sh_sim, sh_msk, all_sim, all_msk,
                   ids_v, rs_v):
    B, P = sim_hbm.shape
    cid = lax.axis_index("c")
    sid = lax.axis_index("s")
    ii = lax.broadcasted_iota(jnp.int32, (16,), 0)

    @pl.when(cid == 0)
    def _():
        pltpu.sync_copy(sim_hbm.at[pl.ds(2 * sid, 2)], myrows)
        for r in range(2):
            v = [myrows[r, pl.ds(16 * g, 16)] for g in range(4)]
            rk = _rank64(v, strict_only=False)
            for g in range(4):
                mymask[pl.ds(64 * r + 16 * g, 16)] = (
                    rk[g] < TOP_K).astype(jnp.int32)
            for g in range(4):
                myflat[pl.ds(64 * r + 16 * g, 16)] = v[g]
        pltpu.sync_copy(myflat, sh_sim.at[sid])
        pltpu.sync_copy(mymask, sh_msk.at[sid])

    plsc.subcore_barrier()

    @pl.when((cid == 0) & (sid == 0))
    def _():
        pltpu.sync_copy(sh_sim, all_sim)
        pltpu.sync_copy(sh_msk, all_msk)
        cnt = [jnp.zeros((16,), jnp.int32) for _ in range(4)]
        col = [jnp.zeros((16,), jnp.float32) for _ in range(4)]
        for t in range(16):
            for r in range(2):
                for g in range(4):
                    cnt[g] = cnt[g] + all_msk[t, pl.ds(64 * r + 16 * g, 16)]
                    col[g] = col[g] + all_sim[t, pl.ds(64 * r + 16 * g, 16)]
        key2 = [cnt[g] * (2 * P) + (P - 1 - (ii + 16 * g)) for g in range(4)]
        rk2 = _rank64(key2, strict_only=True)
        ids_v[...] = jnp.zeros((16,), jnp.int32)
        for g in range(4):
            plsc.store_scatter(ids_v, [rk2[g]], ii + 16 * g, mask=rk2[g] < 16)
        acc = jnp.zeros((16,), jnp.float32)
        for g in range(4):
            acc = acc + jnp.where(rk2[g] < TOP_K, col[g],
                                  jnp.zeros((16,), jnp.float32))
        total_v = (jnp.zeros((16,), jnp.float32) + plsc.cumsum(acc)[15])
        total_v = total_v * jnp.float32(1.0 / B)
        rs_v[...] = jnp.where(ii == 0, total_v,
                              jnp.zeros((16,), jnp.float32))
        pltpu.sync_copy(ids_v, ids_out)
        pltpu.sync_copy(rs_v, rs_out)


def _make_sc_route(B, P):
    mesh = plsc.VectorSubcoreMesh(core_axis_name="c", subcore_axis_name="s")
    return functools.partial(
        pl.kernel,
        mesh=mesh,
        compiler_params=pltpu.CompilerParams(needs_layout_passes=False),
        out_type=(
            jax.ShapeDtypeStruct((16,), jnp.int32),
            jax.ShapeDtypeStruct((16,), jnp.float32),
        ),
        scratch_types=[
            pltpu.VMEM((2, P), jnp.float32),
            pltpu.VMEM((2 * P,), jnp.float32),
            pltpu.VMEM((2 * P,), jnp.int32),
            pltpu.VMEM_SHARED((16, 2 * P), jnp.float32),
            pltpu.VMEM_SHARED((16, 2 * P), jnp.int32),
            pltpu.VMEM((16, 2 * P), jnp.float32),
            pltpu.VMEM((16, 2 * P), jnp.int32),
            pltpu.VMEM((16,), jnp.int32),
            pltpu.VMEM((16,), jnp.float32),
        ],
    )(_sc_route_body)


def _bcast_body(ids_ref, *refs):
    del ids_ref
    out_ref = refs[-1]
    rows = refs[:-1]
    for k, row_ref in enumerate(rows):
        row = row_ref[...]
        LEN, C = row.shape[2], row.shape[3]
        out_ref[:, :, k * LEN:(k + 1) * LEN, :] = jax.lax.broadcast_in_dim(
            row.reshape(LEN, C), (1, out_ref.shape[1], LEN, C), (2, 3))


def kernel(x_embed, cls_features, prompt, prompt_key):
    B = x_embed.shape[0]
    L, P, LEN, C = prompt.shape

    sim = pl.pallas_call(
        _sim_body,
        out_shape=jax.ShapeDtypeStruct((B, P), jnp.float32),
        in_specs=[
            pl.BlockSpec(memory_space=pltpu.VMEM),
            pl.BlockSpec(memory_space=pltpu.VMEM),
        ],
        out_specs=pl.BlockSpec(memory_space=pltpu.VMEM),
    )(cls_features, prompt_key)

    ids16, rs16 = _make_sc_route(B, P)(sim)
    ids = ids16[:TOP_K].reshape(1, TOP_K)
    rs = rs16[0]

    def mk_spec(k):
        return pl.BlockSpec((1, 1, LEN, C),
                            lambda l, ids_s, _k=k: (l, ids_s[0, _k], 0, 0))

    batched_prompt = pl.pallas_call(
        _bcast_body,
        grid_spec=pltpu.PrefetchScalarGridSpec(
            num_scalar_prefetch=1,
            grid=(L,),
            in_specs=[mk_spec(k) for k in range(TOP_K)],
            out_specs=pl.BlockSpec((1, B, TOP_K * LEN, C),
                                   lambda l, ids_s: (l, 0, 0, 0)),
        ),
        out_shape=jax.ShapeDtypeStruct((L, B, TOP_K * LEN, C), jnp.float32),
    )(ids, *([prompt] * TOP_K))

    return batched_prompt, rs


# speedup vs baseline: 3.5475x; 1.0090x over previous
import functools

import jax
import jax.numpy as jnp
from jax import lax
from jax.experimental import pallas as pl
from jax.experimental.pallas import tpu as pltpu
from jax.experimental.pallas import tpu_sc as plsc

TOP_K = 8


def _sim_body(cls_ref, key_ref, sim_ref):
    eps = 1e-12
    k = key_ref[...]
    kn = jnp.sqrt(jnp.sum(k * k, axis=1, keepdims=True))
    k_n = k / jnp.maximum(kn, eps)
    x = cls_ref[...]
    xn = jnp.sqrt(jnp.sum(x * x, axis=1, keepdims=True))
    x_n = x / jnp.maximum(xn, eps)
    sim_ref[...] = jax.lax.dot_general(
        x_n, k_n, (((1,), (1,)), ((), ())),
        precision=jax.lax.Precision.HIGHEST,
        preferred_element_type=jnp.float32)


def _rank64(vregs, strict_only):
    ii = lax.broadcasted_iota(jnp.int32, (16,), 0)
    cnts = [jnp.zeros((16,), jnp.int32) for _ in range(4)]
    for pp in range(64):
        sp = jnp.zeros((16,), vregs[0].dtype) + vregs[pp // 16][pp % 16]
        for g in range(4):
            p_vec = ii + 16 * g
            gtm = (sp > vregs[g]).astype(jnp.int32)
            if strict_only:
                cnts[g] = cnts[g] + gtm
            else:
                eqm = ((sp == vregs[g]) & (pp < p_vec)).astype(jnp.int32)
                cnts[g] = cnts[g] + gtm + eqm
    return cnts


def _sc_route_body(sim_hbm, ids_out, rs_out,
                   myrows, myflat, mymask, sh_sim, sh_msk, all_sim, all_msk,
                   ids_v, rs_v):
    B, P = sim_hbm.shape
    cid = lax.axis_index("c")
    sid = lax.axis_index("s")
    ii = lax.broadcasted_iota(jnp.int32, (16,), 0)

    @pl.when(cid == 0)
    def _():
        pltpu.sync_copy(sim_hbm.at[pl.ds(2 * sid, 2)], myrows)
        for r in range(2):
            v = [myrows[r, pl.ds(16 * g, 16)] for g in range(4)]
            rk = _rank64(v, strict_only=False)
            for g in range(4):
                mymask[pl.ds(64 * r + 16 * g, 16)] = (
                    rk[g] < TOP_K).astype(jnp.int32)
            for g in range(4):
                myflat[pl.ds(64 * r + 16 * g, 16)] = v[g]
        pltpu.sync_copy(myflat, sh_sim.at[sid])
        pltpu.sync_copy(mymask, sh_msk.at[sid])

    plsc.subcore_barrier()

    @pl.when((cid == 0) & (sid == 0))
    def _():
        pltpu.sync_copy(sh_sim, all_sim)
        pltpu.sync_copy(sh_msk, all_msk)
        cnt = [jnp.zeros((16,), jnp.int32) for _ in range(4)]
        col = [jnp.zeros((16,), jnp.float32) for _ in range(4)]
        for t in range(16):
            for r in range(2):
                for g in range(4):
                    cnt[g] = cnt[g] + all_msk[t, pl.ds(64 * r + 16 * g, 16)]
                    col[g] = col[g] + all_sim[t, pl.ds(64 * r + 16 * g, 16)]
        key2 = [cnt[g] * (2 * P) + (P - 1 - (ii + 16 * g)) for g in range(4)]
        rk2 = _rank64(key2, strict_only=True)
        ids_v[...] = jnp.zeros((16,), jnp.int32)
        for g in range(4):
            plsc.store_scatter(ids_v, [rk2[g]], ii + 16 * g, mask=rk2[g] < 16)
        acc = jnp.zeros((16,), jnp.float32)
        for g in range(4):
            acc = acc + jnp.where(rk2[g] < TOP_K, col[g],
                                  jnp.zeros((16,), jnp.float32))
        total_v = (jnp.zeros((16,), jnp.float32) + plsc.cumsum(acc)[15])
        total_v = total_v * jnp.float32(1.0 / B)
        rs_v[...] = jnp.where(ii == 0, total_v,
                              jnp.zeros((16,), jnp.float32))
        pltpu.sync_copy(ids_v, ids_out)
        pltpu.sync_copy(rs_v, rs_out)


def _make_sc_route(B, P):
    mesh = plsc.VectorSubcoreMesh(core_axis_name="c", subcore_axis_name="s")
    return functools.partial(
        pl.kernel,
        mesh=mesh,
        compiler_params=pltpu.CompilerParams(needs_layout_passes=False),
        out_type=(
            jax.ShapeDtypeStruct((16,), jnp.int32),
            jax.ShapeDtypeStruct((16,), jnp.float32),
        ),
        scratch_types=[
            pltpu.VMEM((2, P), jnp.float32),
            pltpu.VMEM((2 * P,), jnp.float32),
            pltpu.VMEM((2 * P,), jnp.int32),
            pltpu.VMEM_SHARED((16, 2 * P), jnp.float32),
            pltpu.VMEM_SHARED((16, 2 * P), jnp.int32),
            pltpu.VMEM((16, 2 * P), jnp.float32),
            pltpu.VMEM((16, 2 * P), jnp.int32),
            pltpu.VMEM((16,), jnp.int32),
            pltpu.VMEM((16,), jnp.float32),
        ],
    )(_sc_route_body)


def _bcast_body(ids_ref, *refs):
    del ids_ref
    out_ref = refs[-1]
    rows = refs[:-1]
    for k, row_ref in enumerate(rows):
        row = row_ref[...]
        LEN, C = row.shape[2], row.shape[3]
        out_ref[:, :, k * LEN:(k + 1) * LEN, :] = jax.lax.broadcast_in_dim(
            row.reshape(LEN, C), (1, out_ref.shape[1], LEN, C), (2, 3))


def kernel(x_embed, cls_features, prompt, prompt_key):
    B = x_embed.shape[0]
    L, P, LEN, C = prompt.shape

    sim = pl.pallas_call(
        _sim_body,
        out_shape=jax.ShapeDtypeStruct((B, P), jnp.float32),
        in_specs=[
            pl.BlockSpec(memory_space=pltpu.VMEM),
            pl.BlockSpec(memory_space=pltpu.VMEM),
        ],
        out_specs=pl.BlockSpec(memory_space=pltpu.VMEM),
    )(cls_features, prompt_key)

    ids16, rs16 = _make_sc_route(B, P)(sim)
    ids = ids16[:TOP_K].reshape(1, TOP_K)
    rs = rs16[0]

    def mk_spec(k):
        return pl.BlockSpec((1, 1, LEN, C),
                            lambda l, ids_s, _k=k: (l, ids_s[0, _k], 0, 0))

    batched_prompt = pl.pallas_call(
        _bcast_body,
        grid_spec=pltpu.PrefetchScalarGridSpec(
            num_scalar_prefetch=1,
            grid=(L,),
            in_specs=[mk_spec(k) for k in range(TOP_K)],
            out_specs=pl.BlockSpec((1, B, TOP_K * LEN, C),
                                   lambda l, ids_s: (l, 0, 0, 0)),
        ),
        out_shape=jax.ShapeDtypeStruct((L, B, TOP_K * LEN, C), jnp.float32),
    )(ids, *([prompt] * TOP_K))

    return batched_prompt, rs
